# baseline (speedup 1.0000x reference)
import functools

import jax
import jax.numpy as jnp
from jax import lax
from jax.experimental import pallas as pl
from jax.experimental.pallas import tpu as pltpu
from jax.experimental.pallas import tpu_sc as plsc


def kernel(x, pos_emb_table):
    B, L, D = x.shape
    NC, NS = 2, 16
    NW = NC * NS
    rows_w = L // NW
    R = 8
    n_chunks = rows_w // R
    n_steps = n_chunks * B
    NB = 4
    SPN = 4
    mesh = plsc.VectorSubcoreMesh(core_axis_name="c", subcore_axis_name="s")

    @functools.partial(
        pl.kernel,
        mesh=mesh,
        out_type=jax.ShapeDtypeStruct((B, L, D), jnp.float32),
        scratch_types=[
            pltpu.VMEM_SHARED((NS, SPN, R, D), jnp.float32),
            pltpu.VMEM((R, D), jnp.float32),
            pltpu.VMEM((R, D), jnp.float32),
            pltpu.VMEM((R, D), jnp.float32),
            pltpu.VMEM((R, D), jnp.float32),
            pltpu.VMEM((R, D), jnp.float32),
            pltpu.VMEM((R, D), jnp.float32),
            pltpu.SemaphoreType.DMA,
            pltpu.SemaphoreType.DMA,
            pltpu.SemaphoreType.DMA,
            pltpu.SemaphoreType.DMA,
            pltpu.SemaphoreType.DMA,
            pltpu.SemaphoreType.DMA,
            pltpu.SemaphoreType.DMA,
            pltpu.SemaphoreType.DMA,
            pltpu.SemaphoreType.DMA,
            pltpu.SemaphoreType.DMA,
            pltpu.SemaphoreType.DMA,
            pltpu.SemaphoreType.DMA,
            pltpu.SemaphoreType.DMA,
            pltpu.SemaphoreType.DMA,
        ],
    )
    def sc_add(x_hbm, t_hbm, out_hbm, sp_x, xv0, xv1, xv2, xv3, tv0, tv1,
               sps0, sps1, sps2, sps3, lds0, lds1, lds2, lds3,
               sts0, sts1, sts2, sts3, tls0, tls1):
        cid = lax.axis_index("c")
        sid = lax.axis_index("s")
        wid = cid * NS + sid
        row0 = wid * rows_w
        xv = (xv0, xv1, xv2, xv3)
        sps = (sps0, sps1, sps2, sps3)
        lds = (lds0, lds1, lds2, lds3)
        sts = (sts0, sts1, sts2, sts3)
        tv = (tv0, tv1)
        tls = (tls0, tls1)

        def start_spload(g):
            c, b = divmod(g, B)
            return pltpu.async_copy(
                x_hbm.at[b, pl.ds(row0 + c * R, R), :],
                sp_x.at[sid, g % SPN], sps[g % SPN])

        def start_xload(g):
            return pltpu.async_copy(
                sp_x.at[sid, g % SPN], xv[g % NB], lds[g % NB])

        def start_tload(c):
            return pltpu.async_copy(
                t_hbm.at[pl.ds(row0 + c * R, R), :], tv[c % 2], tls[c % 2])

        pend_st = [None] * NB
        pend_t = start_tload(0)
        pend_sp = [None] * SPN
        pend_ld = [None] * NB
        for g in range(min(SPN, n_steps)):
            pend_sp[g % SPN] = start_spload(g)
        for g in range(min(2, n_steps)):
            pend_sp[g % SPN].wait()
            pend_sp[g % SPN] = None
            pend_ld[g % NB] = start_xload(g)

        for g in range(n_steps):
            p = g % NB
            c, b = divmod(g, B)
            if b == 0:
                pend_t.wait()
                if c + 1 < n_chunks:
                    pend_t = start_tload(c + 1)
            pend_ld[p].wait()
            t_buf = tv[c % 2]
            x_buf = xv[p]

            def add_body(i, _):
                r = i >> 3
                cb = (i & 7) * 128
                sls = [pl.ds(cb + j * 16, 16) for j in range(8)]
                vals = [t_buf[r, sl] for sl in sls]
                for sl, v in zip(sls, vals):
                    plsc.addupdate(x_buf.at[r, sl], v)
                return 0

            lax.fori_loop(0, R * (D // 128), add_body, 0)
            pend_st[p] = pltpu.async_copy(
                x_buf, out_hbm.at[b, pl.ds(row0 + c * R, R), :], sts[p])
            if g + SPN < n_steps:
                pend_sp[(g + SPN) % SPN] = start_spload(g + SPN)
            if g + 2 < n_steps:
                q = (g + 2) % NB
                if pend_sp[(g + 2) % SPN] is not None:
                    pend_sp[(g + 2) % SPN].wait()
                    pend_sp[(g + 2) % SPN] = None
                if pend_st[q] is not None:
                    pend_st[q].wait()
                    pend_st[q] = None
                pend_ld[q] = start_xload(g + 2)

        for p in range(NB):
            if pend_st[p] is not None:
                pend_st[p].wait()

    return sc_add(x, pos_emb_table)


# speedup vs baseline: 1.0613x; 1.0613x over previous
import functools

import jax
import jax.numpy as jnp
from jax import lax
from jax.experimental import pallas as pl
from jax.experimental.pallas import tpu as pltpu
from jax.experimental.pallas import tpu_sc as plsc


def kernel(x, pos_emb_table):
    B, L, D = x.shape
    NC, NS = 2, 16
    NW = NC * NS
    rows_w = L // NW
    R = 16
    n_chunks = rows_w // R
    n_steps = n_chunks * B
    mesh = plsc.VectorSubcoreMesh(core_axis_name="c", subcore_axis_name="s")

    @functools.partial(
        pl.kernel,
        mesh=mesh,
        out_type=jax.ShapeDtypeStruct((B, L, D), jnp.float32),
        scratch_types=[
            pltpu.VMEM((R, D), jnp.float32),
            pltpu.VMEM((R, D), jnp.float32),
            pltpu.VMEM((R, D), jnp.float32),
            pltpu.VMEM((R, D), jnp.float32),
            pltpu.VMEM((R, D), jnp.float32),
            pltpu.VMEM((R, D), jnp.float32),
            pltpu.VMEM((R, D), jnp.float32),
            pltpu.SemaphoreType.DMA,
            pltpu.SemaphoreType.DMA,
            pltpu.SemaphoreType.DMA,
            pltpu.SemaphoreType.DMA,
            pltpu.SemaphoreType.DMA,
            pltpu.SemaphoreType.DMA,
            pltpu.SemaphoreType.DMA,
            pltpu.SemaphoreType.DMA,
            pltpu.SemaphoreType.DMA,
            pltpu.SemaphoreType.DMA,
            pltpu.SemaphoreType.DMA,
            pltpu.SemaphoreType.DMA,
        ],
    )
    def sc_add(x_hbm, t_hbm, out_hbm, xv0, xv1, xv2, xv3, xv4, tv0, tv1,
               lds0, lds1, lds2, lds3, lds4, sts0, sts1, sts2, sts3, sts4,
               tls0, tls1):
        wid = lax.axis_index("c") * NS + lax.axis_index("s")
        row0 = wid * rows_w
        NB = 5
        xv = (xv0, xv1, xv2, xv3, xv4)
        lds = (lds0, lds1, lds2, lds3, lds4)
        sts = (sts0, sts1, sts2, sts3, sts4)
        tv = (tv0, tv1)
        tls = (tls0, tls1)

        def start_xload(g):
            c, b = divmod(g, B)
            return pltpu.async_copy(
                x_hbm.at[b, pl.ds(row0 + c * R, R), :], xv[g % NB],
                lds[g % NB])

        def start_tload(c):
            return pltpu.async_copy(
                t_hbm.at[pl.ds(row0 + c * R, R), :], tv[c % 2], tls[c % 2])

        pend_st = [None] * NB
        pend_t = start_tload(0)
        pend_ld = [None] * NB
        pend_ld[0] = start_xload(0)
        pend_ld[1] = start_xload(1)
        pend_ld[2] = start_xload(2)

        for g in range(n_steps):
            p = g % NB
            c, b = divmod(g, B)
            if b == 0:
                pend_t.wait()
                if c + 1 < n_chunks:
                    pend_t = start_tload(c + 1)
            pend_ld[p].wait()
            t_buf = tv[c % 2]
            x_buf = xv[p]

            def add_body(i, _):
                r = i >> 3
                cb = (i & 7) * 128
                sls = [pl.ds(cb + j * 16, 16) for j in range(8)]
                vals = [t_buf[r, sl] for sl in sls]
                for sl, v in zip(sls, vals):
                    plsc.addupdate(x_buf.at[r, sl], v)
                return 0

            lax.fori_loop(0, R * (D // 128), add_body, 0)
            pend_st[p] = pltpu.async_copy(
                x_buf, out_hbm.at[b, pl.ds(row0 + c * R, R), :], sts[p])
            if g + 3 < n_steps:
                q = (g + 3) % NB
                if pend_st[q] is not None:
                    pend_st[q].wait()
                    pend_st[q] = None
                pend_ld[q] = start_xload(g + 3)

        for p in range(NB):
            if pend_st[p] is not None:
                pend_st[p].wait()

    return sc_add(x, pos_emb_table)


# speedup vs baseline: 1.0626x; 1.0012x over previous
import functools

import jax
import jax.numpy as jnp
from jax import lax
from jax.experimental import pallas as pl
from jax.experimental.pallas import tpu as pltpu
from jax.experimental.pallas import tpu_sc as plsc


def kernel(x, pos_emb_table):
    B, L, D = x.shape
    NC, NS = 2, 16
    NW = NC * NS
    rows_w = L // NW
    R = 8
    n_steps = rows_w // R
    NB = 3
    mesh = plsc.VectorSubcoreMesh(core_axis_name="c", subcore_axis_name="s")

    @functools.partial(
        pl.kernel,
        mesh=mesh,
        out_type=jax.ShapeDtypeStruct((B, L, D), jnp.float32),
        scratch_types=[
            pltpu.VMEM((B, R, D), jnp.float32),
            pltpu.VMEM((B, R, D), jnp.float32),
            pltpu.VMEM((B, R, D), jnp.float32),
            pltpu.VMEM((R, D), jnp.float32),
            pltpu.VMEM((R, D), jnp.float32),
            pltpu.SemaphoreType.DMA,
            pltpu.SemaphoreType.DMA,
            pltpu.SemaphoreType.DMA,
            pltpu.SemaphoreType.DMA,
            pltpu.SemaphoreType.DMA,
            pltpu.SemaphoreType.DMA,
            pltpu.SemaphoreType.DMA,
            pltpu.SemaphoreType.DMA,
        ],
    )
    def sc_add(x_hbm, t_hbm, out_hbm, xv0, xv1, xv2, tv0, tv1,
               lds0, lds1, lds2, sts0, sts1, sts2, tls0, tls1):
        wid = lax.axis_index("c") * NS + lax.axis_index("s")
        row0 = wid * rows_w
        xv = (xv0, xv1, xv2)
        lds = (lds0, lds1, lds2)
        sts = (sts0, sts1, sts2)
        tv = (tv0, tv1)
        tls = (tls0, tls1)

        def start_xload(c):
            return pltpu.async_copy(
                x_hbm.at[:, pl.ds(row0 + c * R, R), :], xv[c % NB],
                lds[c % NB])

        def start_tload(c):
            return pltpu.async_copy(
                t_hbm.at[pl.ds(row0 + c * R, R), :], tv[c % 2], tls[c % 2])

        pend_st = [None] * NB
        pend_t = start_tload(0)
        pend_ld = [None] * NB
        pend_ld[0] = start_xload(0)
        pend_ld[1] = start_xload(1)

        for c in range(n_steps):
            p = c % NB
            pend_t.wait()
            if c + 1 < n_steps:
                pend_t = start_tload(c + 1)
            pend_ld[p].wait()
            t_buf = tv[c % 2]
            x_buf = xv[p]

            def add_body(i, _):
                b = i >> 6
                r = (i >> 3) & 7
                cb = (i & 7) * 128
                sls = [pl.ds(cb + j * 16, 16) for j in range(8)]
                vals = [t_buf[r, sl] for sl in sls]
                for sl, v in zip(sls, vals):
                    plsc.addupdate(x_buf.at[b, r, sl], v)
                return 0

            lax.fori_loop(0, B * R * (D // 128), add_body, 0)
            pend_st[p] = pltpu.async_copy(
                x_buf, out_hbm.at[:, pl.ds(row0 + c * R, R), :], sts[p])
            if c + 2 < n_steps:
                q = (c + 2) % NB
                if pend_st[q] is not None:
                    pend_st[q].wait()
                    pend_st[q] = None
                pend_ld[q] = start_xload(c + 2)

        for p in range(NB):
            if pend_st[p] is not None:
                pend_st[p].wait()

    return sc_add(x, pos_emb_table)
